# Initial kernel scaffold; baseline (speedup 1.0000x reference)
import jax
import jax.numpy as jnp
from jax.experimental import pallas as pl


def kernel(path_length, bias):
    raise NotImplementedError("write your pallas kernel here")


# speedup vs baseline: 4.0913x; 4.0913x over previous
import functools

import jax
import jax.numpy as jnp
from jax import lax
from jax.experimental import pallas as pl
from jax.experimental.pallas import tpu as pltpu
from jax.experimental.pallas import tpu_sc as plsc

H = 12
NB = 3
S = 2048
N = S * S
NC = 2
NS = 16
NW = NC * NS
PER_W = N // NW
C = 4096
CHUNKS = PER_W // C
L = 16
TAB = 48


def _sc_body(pl_hbm, tab_hbm, out_hbm, in_v, out_v, tab_v):
    wid = lax.axis_index("s") * NC + lax.axis_index("c")
    base = wid * PER_W
    pltpu.sync_copy(tab_hbm, tab_v)

    def chunk_body(g, carry):
        off = base + g * C
        pltpu.sync_copy(pl_hbm.at[pl.ds(off, C)], in_v)

        def vec_body(v, carry2):
            x = in_v[pl.ds(v * L, L)]
            b = jnp.minimum(jnp.maximum(x - 1, 0), 2)
            for h in range(H):
                val = plsc.load_gather(tab_v, [b + (NB * h)])
                out_v[h, pl.ds(v * L, L)] = val
            return carry2

        lax.fori_loop(0, C // L, vec_body, 0, unroll=2)
        for h in range(H):
            pltpu.sync_copy(out_v.at[h], out_hbm.at[h, pl.ds(off, C)])
        return carry

    lax.fori_loop(0, CHUNKS, chunk_body, 0)


_sc_kernel = functools.partial(
    pl.kernel,
    out_type=jax.ShapeDtypeStruct((H, N), jnp.float32),
    mesh=plsc.VectorSubcoreMesh(core_axis_name="c", subcore_axis_name="s"),
    compiler_params=pltpu.CompilerParams(needs_layout_passes=False),
    scratch_types=[
        pltpu.VMEM((C,), jnp.int32),
        pltpu.VMEM((H, C), jnp.float32),
        pltpu.VMEM((TAB,), jnp.float32),
    ],
)(_sc_body)


def kernel(path_length, bias):
    flat = path_length.reshape(N).astype(jnp.int32)
    tab = jnp.pad(bias.reshape(H * NB).astype(jnp.float32), (0, TAB - H * NB))
    out = _sc_kernel(flat, tab)
    return out.reshape(1, H, S, S)


# speedup vs baseline: 4.1946x; 1.0253x over previous
import functools

import jax
import jax.numpy as jnp
from jax import lax
from jax.experimental import pallas as pl
from jax.experimental.pallas import tpu as pltpu
from jax.experimental.pallas import tpu_sc as plsc

H = 12
NB = 3
S = 2048
N = S * S
NC = 2
NS = 16
NW = NC * NS
PER_W = N // NW
C = 2048
CHUNKS = PER_W // C
L = 16
T = 8


def _sc_body(pl_hbm, tab_hbm, out_hbm, in_v, out_v, tab_v, in_sem, out_sem):
    wid = lax.axis_index("s") * NC + lax.axis_index("c")
    base = wid * PER_W
    pltpu.sync_copy(tab_hbm, tab_v)

    pltpu.async_copy(pl_hbm.at[pl.ds(base, C)], in_v.at[pl.ds(0, C)], in_sem)

    def _wait_in():
        pltpu.make_async_copy(
            pl_hbm.at[pl.ds(0, C)], in_v.at[pl.ds(0, C)], in_sem).wait()

    def _wait_out():
        pltpu.make_async_copy(
            out_v.at[0], out_hbm.at[:, pl.ds(0, C)], out_sem).wait()

    def _compute(buf):
        def vec_body(i, carry):
            vb = i * T * L
            bkts = []
            for t in range(T):
                x = in_v[pl.ds(buf * C + vb + t * L, L)]
                bkts.append(jnp.minimum(jnp.maximum(x - 1, 0), 2))
            for h in range(H):
                row = tab_v.at[h]
                for t in range(T):
                    out_v[buf, h, pl.ds(vb + t * L, L)] = plsc.load_gather(
                        row, [bkts[t]])
            return carry
        lax.fori_loop(0, C // (L * T), vec_body, 0)

    def pair_body(k, carry):
        off0 = base + (2 * k) * C
        _wait_in()
        pltpu.async_copy(
            pl_hbm.at[pl.ds(off0 + C, C)], in_v.at[pl.ds(C, C)], in_sem)

        @pl.when(k >= 1)
        def _():
            _wait_out()
        _compute(0)
        pltpu.async_copy(out_v.at[0], out_hbm.at[:, pl.ds(off0, C)], out_sem)

        off1 = off0 + C
        _wait_in()

        @pl.when(k < CHUNKS // 2 - 1)
        def _():
            pltpu.async_copy(
                pl_hbm.at[pl.ds(off1 + C, C)], in_v.at[pl.ds(0, C)], in_sem)

        @pl.when(k >= 1)
        def _():
            _wait_out()
        _compute(1)
        pltpu.async_copy(out_v.at[1], out_hbm.at[:, pl.ds(off1, C)], out_sem)
        return carry

    lax.fori_loop(0, CHUNKS // 2, pair_body, 0)
    _wait_out()
    _wait_out()


_sc_kernel = functools.partial(
    pl.kernel,
    out_type=jax.ShapeDtypeStruct((H, N), jnp.float32),
    mesh=plsc.VectorSubcoreMesh(core_axis_name="c", subcore_axis_name="s"),
    compiler_params=pltpu.CompilerParams(needs_layout_passes=False),
    scratch_types=[
        pltpu.VMEM((2 * C,), jnp.int32),
        pltpu.VMEM((2, H, C), jnp.float32),
        pltpu.VMEM((H, L), jnp.float32),
        pltpu.SemaphoreType.DMA,
        pltpu.SemaphoreType.DMA,
    ],
)(_sc_body)


def kernel(path_length, bias):
    flat = path_length.reshape(N).astype(jnp.int32)
    tab = jnp.pad(bias.astype(jnp.float32), ((0, 0), (0, L - NB)))
    out = _sc_kernel(flat, tab)
    return out.reshape(1, H, S, S)


# speedup vs baseline: 4.5164x; 1.0767x over previous
import functools

import jax
import jax.numpy as jnp
from jax import lax
from jax.experimental import pallas as pl
from jax.experimental.pallas import tpu as pltpu
from jax.experimental.pallas import tpu_sc as plsc

H = 12
NB = 3
S = 2048
N = S * S
NC = 2
NS = 16
NW = NC * NS
PER_W = N // NW
C = 2048
CHUNKS = PER_W // C
L = 16
T = 4
HG = 6


def _sc_body(pl_hbm, tab_hbm, out_hbm, in_v, out_v, tab_v, in_sem, out_sem):
    wid = lax.axis_index("s") * NC + lax.axis_index("c")
    base = wid * PER_W
    pltpu.sync_copy(tab_hbm, tab_v)

    pltpu.async_copy(pl_hbm.at[pl.ds(base, C)], in_v.at[pl.ds(0, C)], in_sem)

    def _wait_in():
        pltpu.make_async_copy(
            pl_hbm.at[pl.ds(0, C)], in_v.at[pl.ds(0, C)], in_sem).wait()

    def _wait_out():
        pltpu.make_async_copy(
            out_v.at[0], out_hbm.at[:, pl.ds(0, C)], out_sem).wait()

    def _compute(buf):
        for h0 in range(0, H, HG):
            splats = [[tab_v[pl.ds((h * NB + b) * L, L)] for b in range(NB)]
                      for h in range(h0, h0 + HG)]

            def vec_body(i, carry, splats=splats, h0=h0):
                vb = i * T * L
                for t in range(T):
                    x = in_v[pl.ds(buf * C + vb + t * L, L)]
                    m0 = x <= 1
                    m1 = x == 2
                    for hh in range(HG):
                        t0, t1, t2 = splats[hh]
                        r = jnp.where(m0, t0, jnp.where(m1, t1, t2))
                        out_v[buf, h0 + hh, pl.ds(vb + t * L, L)] = r
                return carry

            lax.fori_loop(0, C // (L * T), vec_body, 0)

    def pair_body(k, carry):
        off0 = base + (2 * k) * C
        _wait_in()
        pltpu.async_copy(
            pl_hbm.at[pl.ds(off0 + C, C)], in_v.at[pl.ds(C, C)], in_sem)

        @pl.when(k >= 1)
        def _():
            _wait_out()
        _compute(0)
        pltpu.async_copy(out_v.at[0], out_hbm.at[:, pl.ds(off0, C)], out_sem)

        off1 = off0 + C
        _wait_in()

        @pl.when(k < CHUNKS // 2 - 1)
        def _():
            pltpu.async_copy(
                pl_hbm.at[pl.ds(off1 + C, C)], in_v.at[pl.ds(0, C)], in_sem)

        @pl.when(k >= 1)
        def _():
            _wait_out()
        _compute(1)
        pltpu.async_copy(out_v.at[1], out_hbm.at[:, pl.ds(off1, C)], out_sem)
        return carry

    lax.fori_loop(0, CHUNKS // 2, pair_body, 0)
    _wait_out()
    _wait_out()


_sc_kernel = functools.partial(
    pl.kernel,
    out_type=jax.ShapeDtypeStruct((H, N), jnp.float32),
    mesh=plsc.VectorSubcoreMesh(core_axis_name="c", subcore_axis_name="s"),
    compiler_params=pltpu.CompilerParams(needs_layout_passes=False),
    scratch_types=[
        pltpu.VMEM((2 * C,), jnp.int32),
        pltpu.VMEM((2, H, C), jnp.float32),
        pltpu.VMEM((H * NB * L,), jnp.float32),
        pltpu.SemaphoreType.DMA,
        pltpu.SemaphoreType.DMA,
    ],
)(_sc_body)


def kernel(path_length, bias):
    flat = path_length.reshape(N).astype(jnp.int32)
    tab = jnp.broadcast_to(
        bias.astype(jnp.float32)[:, :, None], (H, NB, L)).reshape(H * NB * L)
    out = _sc_kernel(flat, tab)
    return out.reshape(1, H, S, S)


# speedup vs baseline: 37.8413x; 8.3786x over previous
import functools

import jax
import jax.numpy as jnp
from jax import lax
from jax.experimental import pallas as pl
from jax.experimental.pallas import tpu as pltpu
from jax.experimental.pallas import tpu_sc as plsc

H = 12
NB = 3
S = 2048
N = S * S
NC = 2
NS = 16
NW = NC * NS
PER_W = N // NW
C = 2048
CHUNKS = PER_W // C
L = 16
T = 4
HG = 6


def _sc_body(pl_hbm, tab_hbm, out_hbm, in_v, out_v, tab_v, in_sem, out_sem):
    wid = lax.axis_index("s") * NC + lax.axis_index("c")
    base = wid * PER_W
    pltpu.sync_copy(tab_hbm, tab_v)

    pltpu.async_copy(pl_hbm.at[pl.ds(base, C)], in_v.at[pl.ds(0, C)], in_sem)

    def _wait_in():
        pltpu.make_async_copy(
            pl_hbm.at[pl.ds(0, C)], in_v.at[pl.ds(0, C)], in_sem).wait()

    def _wait_out():
        for _ in range(H):
            pltpu.make_async_copy(
                out_v.at[0, 0], out_hbm.at[pl.ds(0, C)], out_sem).wait()

    def _start_out(buf, off):
        for h in range(H):
            pltpu.async_copy(
                out_v.at[buf, h], out_hbm.at[pl.ds(h * N + off, C)], out_sem)

    def _compute(buf):
        for h0 in range(0, H, HG):
            splats = [[tab_v[pl.ds((h * NB + b) * L, L)] for b in range(NB)]
                      for h in range(h0, h0 + HG)]

            def vec_body(i, carry, splats=splats, h0=h0):
                vb = i * T * L
                for t in range(T):
                    x = in_v[pl.ds(buf * C + vb + t * L, L)]
                    m0 = x <= 1
                    m1 = x == 2
                    for hh in range(HG):
                        t0, t1, t2 = splats[hh]
                        r = jnp.where(m0, t0, jnp.where(m1, t1, t2))
                        out_v[buf, h0 + hh, pl.ds(vb + t * L, L)] = r
                return carry

            lax.fori_loop(0, C // (L * T), vec_body, 0)

    def pair_body(k, carry):
        off0 = base + (2 * k) * C
        _wait_in()
        pltpu.async_copy(
            pl_hbm.at[pl.ds(off0 + C, C)], in_v.at[pl.ds(C, C)], in_sem)

        @pl.when(k >= 1)
        def _():
            _wait_out()
        _compute(0)
        _start_out(0, off0)

        off1 = off0 + C
        _wait_in()

        @pl.when(k < CHUNKS // 2 - 1)
        def _():
            pltpu.async_copy(
                pl_hbm.at[pl.ds(off1 + C, C)], in_v.at[pl.ds(0, C)], in_sem)

        @pl.when(k >= 1)
        def _():
            _wait_out()
        _compute(1)
        _start_out(1, off1)
        return carry

    lax.fori_loop(0, CHUNKS // 2, pair_body, 0)
    _wait_out()
    _wait_out()


_sc_kernel = functools.partial(
    pl.kernel,
    out_type=jax.ShapeDtypeStruct((H * N,), jnp.float32),
    mesh=plsc.VectorSubcoreMesh(core_axis_name="c", subcore_axis_name="s"),
    compiler_params=pltpu.CompilerParams(needs_layout_passes=False),
    scratch_types=[
        pltpu.VMEM((2 * C,), jnp.int32),
        pltpu.VMEM((2, H, C), jnp.float32),
        pltpu.VMEM((H * NB * L,), jnp.float32),
        pltpu.SemaphoreType.DMA,
        pltpu.SemaphoreType.DMA,
    ],
)(_sc_body)


def kernel(path_length, bias):
    flat = path_length.reshape(N).astype(jnp.int32)
    tab = jnp.broadcast_to(
        bias.astype(jnp.float32)[:, :, None], (H, NB, L)).reshape(H * NB * L)
    out = _sc_kernel(flat, tab)
    return out.reshape(1, H, S, S)


# speedup vs baseline: 86.1801x; 2.2774x over previous
import functools

import jax
import jax.numpy as jnp
from jax import lax
from jax.experimental import pallas as pl
from jax.experimental.pallas import tpu as pltpu
from jax.experimental.pallas import tpu_sc as plsc

H = 12
NB = 3
S = 2048
NC = 2
NS = 16
NW = NC * NS
ROWS_W = S // NW
RB = 8
CW = 512
CHUNKS = (ROWS_W // RB) * (S // CW)
CPR = S // CW
L = 16
T = 4
HG = 6


def _sc_body(pl_hbm, tab_hbm, out_hbm, in_v, out_v, tab_v, in_sem, out_sem):
    wid = lax.axis_index("s") * NC + lax.axis_index("c")
    row_w = wid * ROWS_W
    pltpu.sync_copy(tab_hbm, tab_v)

    def _chunk_rc(g):
        return row_w + (g // CPR) * RB, (g % CPR) * CW

    def _start_in(g, buf):
        r0, c0 = _chunk_rc(g)
        pltpu.async_copy(
            pl_hbm.at[pl.ds(r0, RB), pl.ds(c0, CW)], in_v.at[buf], in_sem)

    def _wait_in():
        pltpu.make_async_copy(
            pl_hbm.at[pl.ds(0, RB), pl.ds(0, CW)], in_v.at[0], in_sem).wait()

    def _start_out(g, buf):
        r0, c0 = _chunk_rc(g)
        for h in range(H):
            pltpu.async_copy(
                out_v.at[buf, h],
                out_hbm.at[h, pl.ds(r0, RB), pl.ds(c0, CW)], out_sem)

    def _wait_out():
        for _ in range(H):
            pltpu.make_async_copy(
                out_v.at[0, 0],
                out_hbm.at[0, pl.ds(0, RB), pl.ds(0, CW)], out_sem).wait()

    def _compute(buf):
        for h0 in range(0, H, HG):
            splats = [[tab_v[pl.ds((h * NB + b) * L, L)] for b in range(NB)]
                      for h in range(h0, h0 + HG)]

            for s in range(RB):
                def vec_body(i, carry, splats=splats, h0=h0, s=s):
                    vb = i * T * L
                    for t in range(T):
                        x = in_v[buf, s, pl.ds(vb + t * L, L)]
                        m0 = x <= 1
                        m1 = x == 2
                        for hh in range(HG):
                            t0, t1, t2 = splats[hh]
                            r = jnp.where(m0, t0, jnp.where(m1, t1, t2))
                            out_v[buf, h0 + hh, s, pl.ds(vb + t * L, L)] = r
                    return carry

                lax.fori_loop(0, CW // (L * T), vec_body, 0)

    def pair_body(k, carry):
        g0 = 2 * k
        _wait_in()
        _start_in(g0 + 1, 1)

        @pl.when(k >= 1)
        def _():
            _wait_out()
        _compute(0)
        _start_out(g0, 0)

        _wait_in()

        @pl.when(k < CHUNKS // 2 - 1)
        def _():
            _start_in(g0 + 2, 0)

        @pl.when(k >= 1)
        def _():
            _wait_out()
        _compute(1)
        _start_out(g0 + 1, 1)
        return carry

    _start_in(0, 0)
    lax.fori_loop(0, CHUNKS // 2, pair_body, 0)
    _wait_out()
    _wait_out()


_sc_kernel = functools.partial(
    pl.kernel,
    out_type=jax.ShapeDtypeStruct((H, S, S), jnp.float32),
    mesh=plsc.VectorSubcoreMesh(core_axis_name="c", subcore_axis_name="s"),
    compiler_params=pltpu.CompilerParams(
        needs_layout_passes=False, use_tc_tiling_on_sc=True),
    scratch_types=[
        pltpu.VMEM((2, RB, CW), jnp.int32),
        pltpu.VMEM((2, H, RB, CW), jnp.float32),
        pltpu.VMEM((H * NB * L,), jnp.float32),
        pltpu.SemaphoreType.DMA,
        pltpu.SemaphoreType.DMA,
    ],
)(_sc_body)


def kernel(path_length, bias):
    pl2d = path_length.reshape(S, S).astype(jnp.int32)
    tab = jnp.broadcast_to(
        bias.astype(jnp.float32)[:, :, None], (H, NB, L)).reshape(H * NB * L)
    out = _sc_kernel(pl2d, tab)
    return out.reshape(1, H, S, S)
